# Initial kernel scaffold; baseline (speedup 1.0000x reference)
import jax
import jax.numpy as jnp
from jax.experimental import pallas as pl


def kernel(x, dst_init, edge_index, W_conv, b_conv, W_lin, b_lin, W_att, b_att, v_att, W_out, b_out):
    raise NotImplementedError("write your pallas kernel here")


# speedup vs baseline: 4.1941x; 4.1941x over previous
import functools

import jax
import jax.numpy as jnp
from jax import lax
from jax.experimental import pallas as pl
from jax.experimental.pallas import tpu as pltpu
from jax.experimental.pallas import tpu_sc as plsc

N = 50000
H = 64
T = 4
E = 800000

NC, NS = 2, 16
NPAD = 48
NT = N + NPAD
E_PAD = 819200
DEGN = 50176
DEG_SL = DEGN // NS

HALF = N // NC
ACC_R = 25088
ACC_SL = ACC_R // NS

DEG_BLK = 25
SC_BLK = 100
BK = 4


def _mesh():
    return plsc.VectorSubcoreMesh(
        core_axis_name="c", subcore_axis_name="s", num_cores=NC,
        num_subcores=NS)


_SC_PARAMS = pltpu.CompilerParams(use_tc_tiling_on_sc=False)


def _deg_body(e_hbm, ones_hbm, out_hbm, idx_v, ones_v, zbuf, acc_sh, sem):
    c = lax.axis_index("c")
    s = lax.axis_index("s")
    pltpu.sync_copy(ones_hbm, ones_v)
    z16 = jnp.zeros((16,), jnp.float32)

    def zloop(i, carry):
        zbuf[pl.ds(i * 16, 16)] = z16
        return carry

    lax.fori_loop(0, DEG_SL // 16, zloop, 0)
    pltpu.sync_copy(zbuf, acc_sh.at[pl.ds(s * DEG_SL, DEG_SL)])
    plsc.subcore_barrier()

    def blk(j, carry):
        pltpu.sync_copy(e_hbm.at[c, s, j], idx_v)
        for r in range(16):
            pltpu.sync_copy(ones_v.at[r], acc_sh.at[idx_v.at[r]], add=True)
        return carry

    lax.fori_loop(0, DEG_BLK, blk, 0)
    plsc.subcore_barrier()
    pltpu.sync_copy(acc_sh.at[pl.ds(s * DEG_SL, DEG_SL)], zbuf)
    pltpu.sync_copy(zbuf, out_hbm.at[pl.ds(c * DEGN + s * DEG_SL, DEG_SL)])


def _deg_kernel(e_view, ones):
    f = pl.kernel(
        _deg_body,
        out_type=jax.ShapeDtypeStruct((NC * DEGN,), jnp.float32),
        mesh=_mesh(),
        scratch_types=[
            pltpu.VMEM((16, 128), jnp.int32),
            pltpu.VMEM((16, 128), jnp.float32),
            pltpu.VMEM((DEG_SL,), jnp.float32),
            pltpu.VMEM_SHARED((DEGN,), jnp.float32),
            pltpu.SemaphoreType.DMA,
        ],
        compiler_params=_SC_PARAMS,
    )
    return f(e_view, ones)


HW = H // 2


def _scat_body(e_hbm, h_hbm, out_hbm, srcb, dstb, gidx, ldst, rows, acc_sh,
               sem):
    c = lax.axis_index("c")
    s = lax.axis_index("s")
    half_lo = c * HALF
    z16 = jnp.zeros((16,), jnp.float32)
    base = s * ACC_SL

    for t in range(T):
        for fh in range(2):
            def zloop(i, carry):
                for l in range(2):
                    rows[i, pl.ds(l * 16, 16)] = z16
                return carry

            lax.fori_loop(0, BK * 128, zloop, 0)
            for k in range(3):
                pltpu.sync_copy(rows, acc_sh.at[pl.ds(base + k * 512, 512)])
            pltpu.sync_copy(rows.at[pl.ds(0, 32)],
                            acc_sh.at[pl.ds(base + 1536, 32)])
            plsc.subcore_barrier()
            goff = 2 * t * NT + fh

            def blk(j, carry):
                pltpu.sync_copy(e_hbm.at[0, s, j], srcb)
                pltpu.sync_copy(e_hbm.at[1, s, j], dstb)
                for r in range(BK):
                    for l in range(8):
                        sl = (r, pl.ds(l * 16, 16))
                        sv = srcb[sl]
                        dv = dstb[sl]
                        gidx[sl] = sv * 2 + goff
                        ld = dv - half_lo
                        ok = (ld >= 0) & (ld < HALF)
                        garb = HALF + jnp.bitwise_and(dv, 63)
                        ldst[sl] = jnp.where(ok, ld, garb)
                cps = [
                    pltpu.async_copy(h_hbm.at[gidx.at[r]],
                                     rows.at[pl.ds(r * 128, 128)], sem)
                    for r in range(BK)
                ]
                for cp in cps:
                    cp.wait()
                for r in range(BK):
                    pltpu.sync_copy(rows.at[pl.ds(r * 128, 128)],
                                    acc_sh.at[ldst.at[r]], add=True)
                return carry

            lax.fori_loop(0, SC_BLK, blk, 0)
            plsc.subcore_barrier()
            for k in range(3):
                pltpu.sync_copy(acc_sh.at[pl.ds(base + k * 512, 512)], rows)
                pltpu.sync_copy(
                    rows, out_hbm.at[fh, t, c, pl.ds(base + k * 512, 512)])
            pltpu.sync_copy(acc_sh.at[pl.ds(base + 1536, 32)],
                            rows.at[pl.ds(0, 32)])
            pltpu.sync_copy(rows.at[pl.ds(0, 32)],
                            out_hbm.at[fh, t, c, pl.ds(base + 1536, 32)])
            plsc.subcore_barrier()


def _scat_kernel(e_view, h_table):
    f = pl.kernel(
        _scat_body,
        out_type=jax.ShapeDtypeStruct((2, T, NC, ACC_R, HW), jnp.float32),
        mesh=_mesh(),
        scratch_types=[
            pltpu.VMEM((BK, 128), jnp.int32),
            pltpu.VMEM((BK, 128), jnp.int32),
            pltpu.VMEM((BK, 128), jnp.int32),
            pltpu.VMEM((BK, 128), jnp.int32),
            pltpu.VMEM((BK * 128, HW), jnp.float32),
            pltpu.VMEM_SHARED((ACC_R, HW), jnp.float32),
            pltpu.SemaphoreType.DMA,
        ],
        compiler_params=_SC_PARAMS,
    )
    return f(e_view, h_table)


def _conv_body(x_ref, deg_ref, w_ref, out_ref):
    deg = deg_ref[...]
    nrm = jnp.where(deg > 0, lax.rsqrt(jnp.maximum(deg, 1e-12)), 0.0)
    xb = x_ref[:, 0, 0, :]
    out_ref[0] = jnp.dot(xb * nrm, w_ref[...],
                         preferred_element_type=jnp.float32)


def _conv_kernel(x, deg_src, w_conv):
    nb = N // 2000
    return pl.pallas_call(
        _conv_body,
        grid=(nb, T),
        in_specs=[
            pl.BlockSpec((2000, 1, 1, H), lambda i, t: (i, t, 0, 0)),
            pl.BlockSpec((2000, 1), lambda i, t: (i, 0)),
            pl.BlockSpec((H, H), lambda i, t: (0, 0)),
        ],
        out_specs=pl.BlockSpec((1, 2000, H), lambda i, t: (t, i, 0)),
        out_shape=jax.ShapeDtypeStruct((T, NT, H), jnp.float32),
    )(x, deg_src, w_conv)


def _readout_body(agg_ref, degd_ref, prev_ref, wl_ref, bl_ref, bc_ref,
                  wa_ref, ba_ref, va_ref, wo_ref, bo_ref, out_ref):
    deg = degd_ref[...]
    nrm = jnp.where(deg > 0, lax.rsqrt(jnp.maximum(deg, 1e-12)), 0.0)
    prev = prev_ref[...]
    wa = wa_ref[...]
    ba = ba_ref[...]
    va = va_ref[...]
    for t in range(T):
        a = jnp.concatenate([agg_ref[0, t, 0], agg_ref[1, t, 0]],
                            axis=1) * nrm + bc_ref[...]
        z1 = jnp.where(a >= 0, a, 0.01 * a)
        z2 = prev * wl_ref[...] + bl_ref[...]
        s1 = jnp.dot(jnp.tanh(
            jnp.dot(z1, wa, preferred_element_type=jnp.float32) + ba),
            va, preferred_element_type=jnp.float32)
        s2 = jnp.dot(jnp.tanh(
            jnp.dot(z2, wa, preferred_element_type=jnp.float32) + ba),
            va, preferred_element_type=jnp.float32)
        m = jnp.maximum(s1, s2)
        e1 = jnp.exp(s1 - m)
        e2 = jnp.exp(s2 - m)
        al = e1 / (e1 + e2)
        states = al * z1 + (1.0 - al) * z2
        prev = jnp.dot(states, wo_ref[...],
                       preferred_element_type=jnp.float32) + bo_ref[...]
        out_ref[:, t:t + 1] = prev


def _readout_kernel(agg, deg_dst, dst_init, wl, bl, bc, wa, ba, va, wo, bo):
    nbh = HALF // 1000
    return pl.pallas_call(
        _readout_body,
        grid=(NC * nbh,),
        in_specs=[
            pl.BlockSpec((2, T, 1, 1000, HW),
                         lambda i: (0, 0, i // nbh, i % nbh, 0)),
            pl.BlockSpec((1000, 1), lambda i: (i, 0)),
            pl.BlockSpec((1000, 1), lambda i: (i, 0)),
        ] + [pl.BlockSpec(w.shape, lambda i, n=w.ndim: (0,) * n)
             for w in (wl, bl, bc, wa, ba, va, wo, bo)],
        out_specs=pl.BlockSpec((1000, T), lambda i: (i, 0)),
        out_shape=jax.ShapeDtypeStruct((N, T), jnp.float32),
    )(agg, deg_dst, dst_init, wl, bl, bc, wa, ba, va, wo, bo)


def kernel(x, dst_init, edge_index, W_conv, b_conv, W_lin, b_lin, W_att,
           b_att, v_att, W_out, b_out):
    npad = E_PAD - E
    padv = (N + (jnp.arange(npad, dtype=jnp.int32) % NPAD))[None, :]
    e2 = jnp.concatenate(
        [edge_index, jnp.broadcast_to(padv, (2, npad))], axis=1)

    deg_view = e2.reshape(2, NS, DEG_BLK, 16, 128)
    ones = jnp.ones((16, 128), jnp.float32)
    deg = _deg_kernel(deg_view, ones)

    deg_src = deg[:N].reshape(N, 1)
    deg_dst = deg[DEGN:DEGN + N].reshape(N, 1)

    h_table = _conv_kernel(x, deg_src, W_conv).reshape(T * NT * 2, HW)

    scat_view = e2.reshape(2, NS, SC_BLK, BK, 128)
    agg = _scat_kernel(scat_view, h_table)

    out = _readout_kernel(
        agg, deg_dst, dst_init,
        W_lin.reshape(1, H), b_lin.reshape(1, H), b_conv.reshape(1, H),
        W_att, b_att.reshape(1, 2 * H), v_att, W_out, b_out.reshape(1, 1))
    return out[:, :, None]
